# Initial kernel scaffold; baseline (speedup 1.0000x reference)
import jax
import jax.numpy as jnp
from jax.experimental import pallas as pl


def kernel(matrix, branch_index, sample_shape):
    raise NotImplementedError("write your pallas kernel here")


# speedup vs baseline: 9.4453x; 9.4453x over previous
import functools

import numpy as np
import jax
import jax.numpy as jnp
from jax import lax
from jax.experimental import pallas as pl
from jax.experimental.pallas import tpu as pltpu
from jax.experimental.pallas import tpu_sc as plsc

_BATCH = 16384
_ROWS = 256
_COLS = 128

_ROTS0 = (13, 15, 26, 6)
_ROTS1 = (17, 29, 16, 24)


def _tf_block(k1, k2, x1, x2):
    ks2 = k1 ^ k2 ^ np.uint32(0x1BD11BDA)

    def rotl(x, r):
        return (x << np.uint32(r)) | lax.shift_right_logical(x, np.uint32(32 - r))

    def rounds(a, b, rots):
        for r in rots:
            a = a + b
            b = rotl(b, r)
            b = a ^ b
        return a, b

    a, b = x1 + k1, x2 + k2
    a, b = rounds(a, b, _ROTS0)
    a, b = a + k2, b + ks2 + np.uint32(1)
    a, b = rounds(a, b, _ROTS1)
    a, b = a + ks2, b + k1 + np.uint32(2)
    a, b = rounds(a, b, _ROTS0)
    a, b = a + k1, b + k2 + np.uint32(3)
    a, b = rounds(a, b, _ROTS1)
    a, b = a + k2, b + ks2 + np.uint32(4)
    a, b = rounds(a, b, _ROTS0)
    a, b = a + ks2, b + k1 + np.uint32(5)
    return a, b


def _np_tf_block(k1, k2, x1, x2):
    old = np.seterr(over="ignore")
    try:
        k1, k2, x1, x2 = (np.uint32(v) for v in (k1, k2, x1, x2))
        ks2 = np.uint32(k1 ^ k2 ^ np.uint32(0x1BD11BDA))

        def rotl(x, r):
            return np.uint32(np.uint32(x << np.uint32(r))
                             | np.uint32(x >> np.uint32(32 - r)))

        def rounds(a, b, rots):
            for r in rots:
                a = np.uint32(a + b)
                b = rotl(b, r)
                b = np.uint32(a ^ b)
            return a, b

        a, b = np.uint32(x1 + k1), np.uint32(x2 + k2)
        a, b = rounds(a, b, _ROTS0)
        a, b = np.uint32(a + k2), np.uint32(b + ks2 + np.uint32(1))
        a, b = rounds(a, b, _ROTS1)
        a, b = np.uint32(a + ks2), np.uint32(b + k1 + np.uint32(2))
        a, b = rounds(a, b, _ROTS0)
        a, b = np.uint32(a + k1), np.uint32(b + k2 + np.uint32(3))
        a, b = rounds(a, b, _ROTS1)
        a, b = np.uint32(a + k2), np.uint32(b + ks2 + np.uint32(4))
        a, b = rounds(a, b, _ROTS0)
        a, b = np.uint32(a + ks2), np.uint32(b + k1 + np.uint32(5))
        return a, b
    finally:
        np.seterr(**old)


_KA = _np_tf_block(0, 42, 0, 0)
_KB = _np_tf_block(0, 42, 0, 1)


def _subkey(k1, k2, j):
    z = jnp.zeros_like(k1)
    return _tf_block(k1, k2, z, z + np.uint32(j))


def _bits(k1, k2):
    z = jnp.zeros_like(k1)
    b1, b2 = _tf_block(k1, k2, z, z)
    return b1 ^ b2


def _uniform01(bits):
    fb = lax.shift_right_logical(bits, np.uint32(9)) | np.uint32(0x3F800000)
    return lax.bitcast_convert_type(fb, jnp.float32) - np.float32(1.0)


_NORMAL_LO = np.nextafter(np.float32(-1.0), np.float32(0.0), dtype=np.float32)
_NORMAL_SCALE = np.float32(np.float32(1.0) - _NORMAL_LO)


def _erfinv(x):
    w = -jnp.log1p(-x * x)
    lt = w < np.float32(5.0)
    w1 = w - np.float32(2.5)
    p1 = jnp.full_like(x, np.float32(2.81022636e-08))
    for cc in (3.43273939e-07, -3.5233877e-06, -4.39150654e-06, 0.00021858087,
               -0.00125372503, -0.00417768164, 0.246640727, 1.50140941):
        p1 = np.float32(cc) + p1 * w1
    w2 = jnp.sqrt(w) - np.float32(3.0)
    p2 = jnp.full_like(x, np.float32(-0.000200214257))
    for cc in (0.000100950558, 0.00134934322, -0.00367342844, 0.00573950773,
               -0.0076224613, 0.00943887047, 1.00167406, 2.83297682):
        p2 = np.float32(cc) + p2 * w2
    return jnp.where(lt, p1, p2) * x


def _normal(k1, k2):
    u = _uniform01(_bits(k1, k2)) * _NORMAL_SCALE + _NORMAL_LO
    u = jnp.maximum(_NORMAL_LO, u)
    return np.float32(np.sqrt(2.0)) * _erfinv(u)


_N_LANES = np.int32(_ROWS * _COLS)


def _sample_body(ab_ref, idx_ref, out_ref,
                 lk1_ref, lk2_ref, V_ref, act_ref,
                 ik1_ref, ik2_ref, x_ref, v_ref, iact_ref):
    one = np.float32(1.0)
    m = jnp.maximum(ab_ref[...], np.float32(0.0))
    alpha = jnp.clip(m, np.float32(0.1), np.float32(5.0))
    idx = idx_ref[...].astype(jnp.uint32)

    row = lax.broadcasted_iota(jnp.int32, (_ROWS, _COLS), 0)
    in_a = row < (_ROWS // 2)
    k1 = jnp.where(in_a, np.uint32(_KA[0]), np.uint32(_KB[0])).astype(jnp.uint32)
    k2 = jnp.where(in_a, np.uint32(_KA[1]), np.uint32(_KB[1])).astype(jnp.uint32)
    e1, e2 = _tf_block(k1, k2, jnp.zeros_like(idx), idx)

    boost_mask = alpha >= one
    alpha_b = jnp.where(boost_mask, alpha, alpha + one)
    d = alpha_b - np.float32(1.0 / 3.0)
    c = np.float32(1.0 / 3.0) / lax.sqrt(d)

    lk1, lk2 = _subkey(e1, e2, 0)
    sk1, sk2 = _subkey(e1, e2, 1)
    lk1_ref[...] = lk1
    lk2_ref[...] = lk2
    V_ref[...] = jnp.ones_like(alpha)
    act_ref[...] = jnp.ones((_ROWS, _COLS), jnp.int32)

    def outer_body(_):
        lk1 = lk1_ref[...]
        lk2 = lk2_ref[...]
        nk1, nk2 = _subkey(lk1, lk2, 0)
        xk1, xk2 = _subkey(lk1, lk2, 1)
        uk1, uk2 = _subkey(lk1, lk2, 2)

        ik1_ref[...] = xk1
        ik2_ref[...] = xk2
        x_ref[...] = jnp.zeros_like(alpha)
        v_ref[...] = -jnp.ones_like(alpha)
        iact_ref[...] = jnp.ones((_ROWS, _COLS), jnp.int32)

        def inner_body(_):
            ik1 = ik1_ref[...]
            ik2 = ik2_ref[...]
            iact = iact_ref[...] != 0
            n1, n2 = _subkey(ik1, ik2, 0)
            ns1, ns2 = _subkey(ik1, ik2, 1)
            xn = _normal(ns1, ns2)
            vn = one + xn * c
            ik1_ref[...] = jnp.where(iact, n1, ik1)
            ik2_ref[...] = jnp.where(iact, n2, ik2)
            x_ref[...] = jnp.where(iact, xn, x_ref[...])
            v_new = jnp.where(iact, vn, v_ref[...])
            v_ref[...] = v_new
            iact = iact & (v_new <= np.float32(0.0))
            iact_ref[...] = iact.astype(jnp.int32)
            return jnp.sum(iact.astype(jnp.int32))

        lax.while_loop(lambda nn: nn > 0, inner_body, _N_LANES)

        x = x_ref[...]
        v = v_ref[...]
        Xn = x * x
        Vn = (v * v) * v
        Un = _uniform01(_bits(uk1, uk2))

        active = act_ref[...] != 0
        lk1_ref[...] = jnp.where(active, nk1, lk1)
        lk2_ref[...] = jnp.where(active, nk2, lk2)
        V_ref[...] = jnp.where(active, Vn, V_ref[...])
        cond = (Un >= one - np.float32(0.0331) * (Xn * Xn)) & (
            lax.log(Un) >= Xn * np.float32(0.5)
            + d * ((one - Vn) + lax.log(Vn)))
        active = active & cond
        act_ref[...] = active.astype(jnp.int32)
        return jnp.sum(active.astype(jnp.int32))

    lax.while_loop(lambda n: n > 0, outer_body, _N_LANES)

    u = _uniform01(_bits(sk1, sk2))
    log_samples = lax.log1p(-u)
    log_boost = jnp.where(boost_mask | (log_samples == np.float32(0.0)),
                          np.float32(0.0),
                          log_samples * (one / alpha))
    lg = (lax.log(d) + lax.log(V_ref[...])) + log_boost

    lga = lg[: _ROWS // 2]
    lgb = lg[_ROWS // 2:]
    log_max = jnp.maximum(lga, lgb)
    ga = jnp.exp(lga - log_max)
    gb = jnp.exp(lgb - log_max)
    out_ref[...] = ga / (ga + gb)


def _tc_sample(ab, idx):
    u32s = functools.partial(pltpu.VMEM, (_ROWS, _COLS))
    return pl.pallas_call(
        _sample_body,
        out_shape=jax.ShapeDtypeStruct((_ROWS // 2, _COLS), jnp.float32),
        scratch_shapes=[
            u32s(jnp.uint32), u32s(jnp.uint32),
            u32s(jnp.float32), u32s(jnp.int32),
            u32s(jnp.uint32), u32s(jnp.uint32),
            u32s(jnp.float32), u32s(jnp.float32),
            u32s(jnp.int32),
        ],
    )(ab, idx)


_NC, _NS = 2, 16
_NW = _NC * _NS
_BPW = _BATCH // _NW
_CH = 128
_NCH = _BPW // _CH


def _sc_gather(matrix, idx):
    mesh = plsc.VectorSubcoreMesh(core_axis_name="c", subcore_axis_name="s")

    @functools.partial(
        pl.kernel, mesh=mesh,
        compiler_params=pltpu.CompilerParams(use_tc_tiling_on_sc=False),
        out_type=jax.ShapeDtypeStruct((_BATCH, 2), jnp.float32),
        scratch_types=[
            pltpu.VMEM((_BPW,), jnp.int32),
            pltpu.VMEM((_BPW, 2), jnp.float32),
            pltpu.SemaphoreType.DMA,
        ],
    )
    def gather_k(table_hbm, idx_hbm, out_hbm, idx_v, rows_v, sem):
        wid = lax.axis_index("s") * _NC + lax.axis_index("c")
        base = wid * _BPW
        pltpu.sync_copy(idx_hbm.at[pl.ds(base, _BPW)], idx_v)
        copies = [
            pltpu.async_copy(
                table_hbm.at[idx_v.at[pl.ds(ci * _CH, _CH)]],
                rows_v.at[pl.ds(ci * _CH, _CH)], sem)
            for ci in range(_NCH)
        ]
        for cp in copies:
            cp.wait()
        pltpu.sync_copy(rows_v, out_hbm.at[pl.ds(base, _BPW)])

    return gather_k(matrix, idx)


def kernel(matrix, branch_index, sample_shape):
    g = _sc_gather(matrix, branch_index.astype(jnp.int32))
    ab = jnp.concatenate([g[:, 0], g[:, 1]]).reshape(_ROWS, _COLS)
    idx2 = jnp.concatenate([branch_index, branch_index]).reshape(_ROWS, _COLS)
    beta = _tc_sample(ab, idx2.astype(jnp.int32)).reshape(_BATCH)
    return beta / jnp.asarray(sample_shape).astype(jnp.float32)


# speedup vs baseline: 11.6027x; 1.2284x over previous
import functools

import numpy as np
import jax
import jax.numpy as jnp
from jax import lax
from jax.experimental import pallas as pl
from jax.experimental.pallas import tpu as pltpu
from jax.experimental.pallas import tpu_sc as plsc

_BATCH = 16384
_ROWS = 256
_COLS = 128

_ROTS0 = (13, 15, 26, 6)
_ROTS1 = (17, 29, 16, 24)


def _tf_block(k1, k2, x1, x2):
    ks2 = k1 ^ k2 ^ np.uint32(0x1BD11BDA)

    def rotl(x, r):
        return (x << np.uint32(r)) | lax.shift_right_logical(x, np.uint32(32 - r))

    def rounds(a, b, rots):
        for r in rots:
            a = a + b
            b = rotl(b, r)
            b = a ^ b
        return a, b

    a, b = x1 + k1, x2 + k2
    a, b = rounds(a, b, _ROTS0)
    a, b = a + k2, b + ks2 + np.uint32(1)
    a, b = rounds(a, b, _ROTS1)
    a, b = a + ks2, b + k1 + np.uint32(2)
    a, b = rounds(a, b, _ROTS0)
    a, b = a + k1, b + k2 + np.uint32(3)
    a, b = rounds(a, b, _ROTS1)
    a, b = a + k2, b + ks2 + np.uint32(4)
    a, b = rounds(a, b, _ROTS0)
    a, b = a + ks2, b + k1 + np.uint32(5)
    return a, b


def _np_tf_block(k1, k2, x1, x2):
    old = np.seterr(over="ignore")
    try:
        k1, k2, x1, x2 = (np.uint32(v) for v in (k1, k2, x1, x2))
        ks2 = np.uint32(k1 ^ k2 ^ np.uint32(0x1BD11BDA))

        def rotl(x, r):
            return np.uint32(np.uint32(x << np.uint32(r))
                             | np.uint32(x >> np.uint32(32 - r)))

        def rounds(a, b, rots):
            for r in rots:
                a = np.uint32(a + b)
                b = rotl(b, r)
                b = np.uint32(a ^ b)
            return a, b

        a, b = np.uint32(x1 + k1), np.uint32(x2 + k2)
        a, b = rounds(a, b, _ROTS0)
        a, b = np.uint32(a + k2), np.uint32(b + ks2 + np.uint32(1))
        a, b = rounds(a, b, _ROTS1)
        a, b = np.uint32(a + ks2), np.uint32(b + k1 + np.uint32(2))
        a, b = rounds(a, b, _ROTS0)
        a, b = np.uint32(a + k1), np.uint32(b + k2 + np.uint32(3))
        a, b = rounds(a, b, _ROTS1)
        a, b = np.uint32(a + k2), np.uint32(b + ks2 + np.uint32(4))
        a, b = rounds(a, b, _ROTS0)
        a, b = np.uint32(a + ks2), np.uint32(b + k1 + np.uint32(5))
        return a, b
    finally:
        np.seterr(**old)


_KA = _np_tf_block(0, 42, 0, 0)
_KB = _np_tf_block(0, 42, 0, 1)


def _subkey(k1, k2, j):
    z = jnp.zeros_like(k1)
    return _tf_block(k1, k2, z, z + np.uint32(j))


def _bits(k1, k2):
    z = jnp.zeros_like(k1)
    b1, b2 = _tf_block(k1, k2, z, z)
    return b1 ^ b2


def _uniform01(bits):
    fb = lax.shift_right_logical(bits, np.uint32(9)) | np.uint32(0x3F800000)
    return lax.bitcast_convert_type(fb, jnp.float32) - np.float32(1.0)


_NORMAL_LO = np.nextafter(np.float32(-1.0), np.float32(0.0), dtype=np.float32)
_NORMAL_SCALE = np.float32(np.float32(1.0) - _NORMAL_LO)


def _erfinv(x):
    w = -jnp.log1p(-x * x)
    lt = w < np.float32(5.0)
    w1 = w - np.float32(2.5)
    p1 = jnp.full_like(x, np.float32(2.81022636e-08))
    for cc in (3.43273939e-07, -3.5233877e-06, -4.39150654e-06, 0.00021858087,
               -0.00125372503, -0.00417768164, 0.246640727, 1.50140941):
        p1 = np.float32(cc) + p1 * w1
    w2 = jnp.sqrt(w) - np.float32(3.0)
    p2 = jnp.full_like(x, np.float32(-0.000200214257))
    for cc in (0.000100950558, 0.00134934322, -0.00367342844, 0.00573950773,
               -0.0076224613, 0.00943887047, 1.00167406, 2.83297682):
        p2 = np.float32(cc) + p2 * w2
    return jnp.where(lt, p1, p2) * x


def _normal(k1, k2):
    u = _uniform01(_bits(k1, k2)) * _NORMAL_SCALE + _NORMAL_LO
    u = jnp.maximum(_NORMAL_LO, u)
    return np.float32(np.sqrt(2.0)) * _erfinv(u)


_N_LANES = np.int32(_ROWS * _COLS)


def _sample_body(ab_ref, idx_ref, out_ref,
                 lk1_ref, lk2_ref, V_ref, act_ref,
                 ik1_ref, ik2_ref, x_ref, v_ref, iact_ref):
    one = np.float32(1.0)
    m = jnp.maximum(ab_ref[...], np.float32(0.0))
    alpha = jnp.clip(m, np.float32(0.1), np.float32(5.0))
    idx = idx_ref[...].astype(jnp.uint32)

    row = lax.broadcasted_iota(jnp.int32, (_ROWS, _COLS), 0)
    in_a = row < (_ROWS // 2)
    k1 = jnp.where(in_a, np.uint32(_KA[0]), np.uint32(_KB[0])).astype(jnp.uint32)
    k2 = jnp.where(in_a, np.uint32(_KA[1]), np.uint32(_KB[1])).astype(jnp.uint32)
    e1, e2 = _tf_block(k1, k2, jnp.zeros_like(idx), idx)

    boost_mask = alpha >= one
    alpha_b = jnp.where(boost_mask, alpha, alpha + one)
    d = alpha_b - np.float32(1.0 / 3.0)
    c = np.float32(1.0 / 3.0) / lax.sqrt(d)

    lk1, lk2 = _subkey(e1, e2, 0)
    sk1, sk2 = _subkey(e1, e2, 1)
    lk1_ref[...] = lk1
    lk2_ref[...] = lk2
    V_ref[...] = jnp.ones_like(alpha)
    act_ref[...] = jnp.ones((_ROWS, _COLS), jnp.int32)

    def outer_body(_):
        lk1 = lk1_ref[...]
        lk2 = lk2_ref[...]
        nk1, nk2 = _subkey(lk1, lk2, 0)
        xk1, xk2 = _subkey(lk1, lk2, 1)
        uk1, uk2 = _subkey(lk1, lk2, 2)

        ik1_ref[...] = xk1
        ik2_ref[...] = xk2
        x_ref[...] = jnp.zeros_like(alpha)
        v_ref[...] = -jnp.ones_like(alpha)
        iact_ref[...] = jnp.ones((_ROWS, _COLS), jnp.int32)

        def inner_body(_):
            ik1 = ik1_ref[...]
            ik2 = ik2_ref[...]
            iact = iact_ref[...] != 0
            n1, n2 = _subkey(ik1, ik2, 0)
            ns1, ns2 = _subkey(ik1, ik2, 1)
            xn = _normal(ns1, ns2)
            vn = one + xn * c
            ik1_ref[...] = jnp.where(iact, n1, ik1)
            ik2_ref[...] = jnp.where(iact, n2, ik2)
            x_ref[...] = jnp.where(iact, xn, x_ref[...])
            v_new = jnp.where(iact, vn, v_ref[...])
            v_ref[...] = v_new
            iact = iact & (v_new <= np.float32(0.0))
            iact_ref[...] = iact.astype(jnp.int32)
            return jnp.sum(iact.astype(jnp.int32))

        lax.while_loop(lambda nn: nn > 0, inner_body, _N_LANES)

        x = x_ref[...]
        v = v_ref[...]
        Xn = x * x
        Vn = (v * v) * v
        Un = _uniform01(_bits(uk1, uk2))

        active = act_ref[...] != 0
        lk1_ref[...] = jnp.where(active, nk1, lk1)
        lk2_ref[...] = jnp.where(active, nk2, lk2)
        V_ref[...] = jnp.where(active, Vn, V_ref[...])
        cond = (Un >= one - np.float32(0.0331) * (Xn * Xn)) & (
            lax.log(Un) >= Xn * np.float32(0.5)
            + d * ((one - Vn) + lax.log(Vn)))
        active = active & cond
        act_ref[...] = active.astype(jnp.int32)
        return jnp.sum(active.astype(jnp.int32))

    lax.while_loop(lambda n: n > 0, outer_body, _N_LANES)

    u = _uniform01(_bits(sk1, sk2))
    log_samples = lax.log1p(-u)
    log_boost = jnp.where(boost_mask | (log_samples == np.float32(0.0)),
                          np.float32(0.0),
                          log_samples * (one / alpha))
    lg = (lax.log(d) + lax.log(V_ref[...])) + log_boost

    lga = lg[: _ROWS // 2]
    lgb = lg[_ROWS // 2:]
    log_max = jnp.maximum(lga, lgb)
    ga = jnp.exp(lga - log_max)
    gb = jnp.exp(lgb - log_max)
    out_ref[...] = ga / (ga + gb)


def _tc_sample(ab, idx):
    u32s = functools.partial(pltpu.VMEM, (_ROWS, _COLS))
    return pl.pallas_call(
        _sample_body,
        out_shape=jax.ShapeDtypeStruct((_ROWS // 2, _COLS), jnp.float32),
        scratch_shapes=[
            u32s(jnp.uint32), u32s(jnp.uint32),
            u32s(jnp.float32), u32s(jnp.int32),
            u32s(jnp.uint32), u32s(jnp.uint32),
            u32s(jnp.float32), u32s(jnp.float32),
            u32s(jnp.int32),
        ],
    )(ab, idx)


_NC, _NS = 2, 16
_NW = _NC * _NS
_BPW = _BATCH // _NW
_PPW = 2 * _BPW
_CH = 128
_LANES = 16


def _sc_gather(tab_flat, idx):
    mesh = plsc.VectorSubcoreMesh(core_axis_name="c", subcore_axis_name="s")

    @functools.partial(
        pl.kernel, mesh=mesh,
        compiler_params=pltpu.CompilerParams(use_tc_tiling_on_sc=False,
                                             needs_layout_passes=False),
        out_type=jax.ShapeDtypeStruct((2 * _BATCH,), jnp.float32),
        scratch_types=[
            pltpu.VMEM((_BPW,), jnp.int32),
            pltpu.VMEM((_PPW,), jnp.int32),
            pltpu.VMEM((_PPW,), jnp.float32),
            pltpu.SemaphoreType.DMA,
        ],
    )
    def gather_k(tab_hbm, idx_hbm, out_hbm, idx_v, iidx_v, pairs_v, sem):
        wid = lax.axis_index("s") * _NC + lax.axis_index("c")
        base = wid * _BPW
        pltpu.sync_copy(idx_hbm.at[pl.ds(base, _BPW)], idx_v)
        lane = lax.iota(jnp.int32, _LANES)
        for j in range(_BPW // _LANES):
            e = idx_v[pl.ds(j * _LANES, _LANES)] << 1
            pos = (lane << 1) + (2 * j * _LANES)
            plsc.store_scatter(iidx_v, [pos], e)
            plsc.store_scatter(iidx_v, [pos + 1], e + 1)
        copies = [
            pltpu.async_copy(
                tab_hbm.at[iidx_v.at[pl.ds(ci * _CH, _CH)]],
                pairs_v.at[pl.ds(ci * _CH, _CH)], sem)
            for ci in range(_PPW // _CH)
        ]
        for cp in copies:
            cp.wait()
        pltpu.sync_copy(pairs_v, out_hbm.at[pl.ds(2 * base, _PPW)])

    return gather_k(tab_flat, idx)


def kernel(matrix, branch_index, sample_shape):
    flat = _sc_gather(matrix.reshape(2 * matrix.shape[0]),
                      branch_index.astype(jnp.int32))
    g = flat.reshape(_BATCH, 2)
    ab = jnp.concatenate([g[:, 0], g[:, 1]]).reshape(_ROWS, _COLS)
    idx2 = jnp.concatenate([branch_index, branch_index]).reshape(_ROWS, _COLS)
    beta = _tc_sample(ab, idx2.astype(jnp.int32)).reshape(_BATCH)
    return beta / jnp.asarray(sample_shape).astype(jnp.float32)


# speedup vs baseline: 173.1529x; 14.9234x over previous
import functools

import numpy as np
import jax
import jax.numpy as jnp
from jax import lax
from jax.experimental import pallas as pl
from jax.experimental.pallas import tpu as pltpu
from jax.experimental.pallas import tpu_sc as plsc

_BATCH = 16384
_ROWS = 256
_COLS = 128

_ROTS0 = (13, 15, 26, 6)
_ROTS1 = (17, 29, 16, 24)


def _tf_block(k1, k2, x1, x2):
    ks2 = k1 ^ k2 ^ np.uint32(0x1BD11BDA)

    def rotl(x, r):
        return (x << np.uint32(r)) | lax.shift_right_logical(x, np.uint32(32 - r))

    def rounds(a, b, rots):
        for r in rots:
            a = a + b
            b = rotl(b, r)
            b = a ^ b
        return a, b

    a, b = x1 + k1, x2 + k2
    a, b = rounds(a, b, _ROTS0)
    a, b = a + k2, b + ks2 + np.uint32(1)
    a, b = rounds(a, b, _ROTS1)
    a, b = a + ks2, b + k1 + np.uint32(2)
    a, b = rounds(a, b, _ROTS0)
    a, b = a + k1, b + k2 + np.uint32(3)
    a, b = rounds(a, b, _ROTS1)
    a, b = a + k2, b + ks2 + np.uint32(4)
    a, b = rounds(a, b, _ROTS0)
    a, b = a + ks2, b + k1 + np.uint32(5)
    return a, b


def _np_tf_block(k1, k2, x1, x2):
    old = np.seterr(over="ignore")
    try:
        k1, k2, x1, x2 = (np.uint32(v) for v in (k1, k2, x1, x2))
        ks2 = np.uint32(k1 ^ k2 ^ np.uint32(0x1BD11BDA))

        def rotl(x, r):
            return np.uint32(np.uint32(x << np.uint32(r))
                             | np.uint32(x >> np.uint32(32 - r)))

        def rounds(a, b, rots):
            for r in rots:
                a = np.uint32(a + b)
                b = rotl(b, r)
                b = np.uint32(a ^ b)
            return a, b

        a, b = np.uint32(x1 + k1), np.uint32(x2 + k2)
        a, b = rounds(a, b, _ROTS0)
        a, b = np.uint32(a + k2), np.uint32(b + ks2 + np.uint32(1))
        a, b = rounds(a, b, _ROTS1)
        a, b = np.uint32(a + ks2), np.uint32(b + k1 + np.uint32(2))
        a, b = rounds(a, b, _ROTS0)
        a, b = np.uint32(a + k1), np.uint32(b + k2 + np.uint32(3))
        a, b = rounds(a, b, _ROTS1)
        a, b = np.uint32(a + k2), np.uint32(b + ks2 + np.uint32(4))
        a, b = rounds(a, b, _ROTS0)
        a, b = np.uint32(a + ks2), np.uint32(b + k1 + np.uint32(5))
        return a, b
    finally:
        np.seterr(**old)


_KA = _np_tf_block(0, 42, 0, 0)
_KB = _np_tf_block(0, 42, 0, 1)


def _subkey(k1, k2, j):
    z = jnp.zeros_like(k1)
    return _tf_block(k1, k2, z, z + np.uint32(j))


def _bits(k1, k2):
    z = jnp.zeros_like(k1)
    b1, b2 = _tf_block(k1, k2, z, z)
    return b1 ^ b2


def _uniform01(bits):
    fb = lax.shift_right_logical(bits, np.uint32(9)) | np.uint32(0x3F800000)
    return lax.bitcast_convert_type(fb, jnp.float32) - np.float32(1.0)


_NORMAL_LO = np.nextafter(np.float32(-1.0), np.float32(0.0), dtype=np.float32)
_NORMAL_SCALE = np.float32(np.float32(1.0) - _NORMAL_LO)


def _erfinv(x):
    w = -jnp.log1p(-x * x)
    lt = w < np.float32(5.0)
    w1 = w - np.float32(2.5)
    p1 = jnp.full_like(x, np.float32(2.81022636e-08))
    for cc in (3.43273939e-07, -3.5233877e-06, -4.39150654e-06, 0.00021858087,
               -0.00125372503, -0.00417768164, 0.246640727, 1.50140941):
        p1 = np.float32(cc) + p1 * w1
    w2 = jnp.sqrt(w) - np.float32(3.0)
    p2 = jnp.full_like(x, np.float32(-0.000200214257))
    for cc in (0.000100950558, 0.00134934322, -0.00367342844, 0.00573950773,
               -0.0076224613, 0.00943887047, 1.00167406, 2.83297682):
        p2 = np.float32(cc) + p2 * w2
    return jnp.where(lt, p1, p2) * x


def _normal(k1, k2):
    u = _uniform01(_bits(k1, k2)) * _NORMAL_SCALE + _NORMAL_LO
    u = jnp.maximum(_NORMAL_LO, u)
    return np.float32(np.sqrt(2.0)) * _erfinv(u)


_N_LANES = np.int32(_ROWS * _COLS)


def _sample_body(ab_ref, idx_ref, out_ref,
                 lk1_ref, lk2_ref, V_ref, act_ref,
                 ik1_ref, ik2_ref, x_ref, v_ref, iact_ref):
    one = np.float32(1.0)
    m = jnp.maximum(ab_ref[...], np.float32(0.0))
    alpha = jnp.clip(m, np.float32(0.1), np.float32(5.0))
    idx = idx_ref[...].astype(jnp.uint32)

    row = lax.broadcasted_iota(jnp.int32, (_ROWS, _COLS), 0)
    in_a = row < (_ROWS // 2)
    k1 = jnp.where(in_a, np.uint32(_KA[0]), np.uint32(_KB[0])).astype(jnp.uint32)
    k2 = jnp.where(in_a, np.uint32(_KA[1]), np.uint32(_KB[1])).astype(jnp.uint32)
    e1, e2 = _tf_block(k1, k2, jnp.zeros_like(idx), idx)

    boost_mask = alpha >= one
    alpha_b = jnp.where(boost_mask, alpha, alpha + one)
    d = alpha_b - np.float32(1.0 / 3.0)
    c = np.float32(1.0 / 3.0) / lax.sqrt(d)

    lk1, lk2 = _subkey(e1, e2, 0)
    sk1, sk2 = _subkey(e1, e2, 1)
    lk1_ref[...] = lk1
    lk2_ref[...] = lk2
    V_ref[...] = jnp.ones_like(alpha)
    act_ref[...] = jnp.ones((_ROWS, _COLS), jnp.int32)

    def outer_body(_):
        lk1 = lk1_ref[...]
        lk2 = lk2_ref[...]
        nk1, nk2 = _subkey(lk1, lk2, 0)
        xk1, xk2 = _subkey(lk1, lk2, 1)
        uk1, uk2 = _subkey(lk1, lk2, 2)

        ik1_ref[...] = xk1
        ik2_ref[...] = xk2
        x_ref[...] = jnp.zeros_like(alpha)
        v_ref[...] = -jnp.ones_like(alpha)
        iact_ref[...] = jnp.ones((_ROWS, _COLS), jnp.int32)

        def inner_body(_):
            ik1 = ik1_ref[...]
            ik2 = ik2_ref[...]
            iact = iact_ref[...] != 0
            n1, n2 = _subkey(ik1, ik2, 0)
            ns1, ns2 = _subkey(ik1, ik2, 1)
            xn = _normal(ns1, ns2)
            vn = one + xn * c
            ik1_ref[...] = jnp.where(iact, n1, ik1)
            ik2_ref[...] = jnp.where(iact, n2, ik2)
            x_ref[...] = jnp.where(iact, xn, x_ref[...])
            v_new = jnp.where(iact, vn, v_ref[...])
            v_ref[...] = v_new
            iact = iact & (v_new <= np.float32(0.0))
            iact_ref[...] = iact.astype(jnp.int32)
            return jnp.sum(iact.astype(jnp.int32))

        lax.while_loop(lambda nn: nn > 0, inner_body, _N_LANES)

        x = x_ref[...]
        v = v_ref[...]
        Xn = x * x
        Vn = (v * v) * v
        Un = _uniform01(_bits(uk1, uk2))

        active = act_ref[...] != 0
        lk1_ref[...] = jnp.where(active, nk1, lk1)
        lk2_ref[...] = jnp.where(active, nk2, lk2)
        V_ref[...] = jnp.where(active, Vn, V_ref[...])
        cond = (Un >= one - np.float32(0.0331) * (Xn * Xn)) & (
            lax.log(Un) >= Xn * np.float32(0.5)
            + d * ((one - Vn) + lax.log(Vn)))
        active = active & cond
        act_ref[...] = active.astype(jnp.int32)
        return jnp.sum(active.astype(jnp.int32))

    lax.while_loop(lambda n: n > 0, outer_body, _N_LANES)

    u = _uniform01(_bits(sk1, sk2))
    log_samples = lax.log1p(-u)
    log_boost = jnp.where(boost_mask | (log_samples == np.float32(0.0)),
                          np.float32(0.0),
                          log_samples * (one / alpha))
    lg = (lax.log(d) + lax.log(V_ref[...])) + log_boost

    lga = lg[: _ROWS // 2]
    lgb = lg[_ROWS // 2:]
    log_max = jnp.maximum(lga, lgb)
    ga = jnp.exp(lga - log_max)
    gb = jnp.exp(lgb - log_max)
    out_ref[...] = ga / (ga + gb)


def _tc_sample(ab, idx):
    u32s = functools.partial(pltpu.VMEM, (_ROWS, _COLS))
    return pl.pallas_call(
        _sample_body,
        out_shape=jax.ShapeDtypeStruct((_ROWS // 2, _COLS), jnp.float32),
        scratch_shapes=[
            u32s(jnp.uint32), u32s(jnp.uint32),
            u32s(jnp.float32), u32s(jnp.int32),
            u32s(jnp.uint32), u32s(jnp.uint32),
            u32s(jnp.float32), u32s(jnp.float32),
            u32s(jnp.int32),
        ],
    )(ab, idx)


_NC, _NS = 2, 16
_NW = _NC * _NS
_BPW = _BATCH // _NW
_PPW = 2 * _BPW
_CH = 128
_LANES = 16


def _sc_gather(acol, bcol, idx):
    mesh = plsc.VectorSubcoreMesh(core_axis_name="c", subcore_axis_name="s")

    @functools.partial(
        pl.kernel, mesh=mesh,
        compiler_params=pltpu.CompilerParams(use_tc_tiling_on_sc=False),
        out_type=jax.ShapeDtypeStruct((2 * _BATCH,), jnp.float32),
        scratch_types=[
            pltpu.VMEM((_BPW,), jnp.int32),
            pltpu.VMEM((_BPW,), jnp.float32),
            pltpu.VMEM((_BPW,), jnp.float32),
            pltpu.SemaphoreType.DMA,
        ],
    )
    def gather_k(a_hbm, b_hbm, idx_hbm, out_hbm, idx_v, a_v, b_v, sem):
        wid = lax.axis_index("s") * _NC + lax.axis_index("c")
        base = wid * _BPW
        pltpu.sync_copy(idx_hbm.at[pl.ds(base, _BPW)], idx_v)
        copies = []
        for ci in range(_BPW // _CH):
            isl = idx_v.at[pl.ds(ci * _CH, _CH)]
            copies.append(pltpu.async_copy(
                a_hbm.at[isl], a_v.at[pl.ds(ci * _CH, _CH)], sem))
            copies.append(pltpu.async_copy(
                b_hbm.at[isl], b_v.at[pl.ds(ci * _CH, _CH)], sem))
        for cp in copies:
            cp.wait()
        pltpu.sync_copy(a_v, out_hbm.at[pl.ds(base, _BPW)])
        pltpu.sync_copy(b_v, out_hbm.at[pl.ds(_BATCH + base, _BPW)])

    return gather_k(acol, bcol, idx)


def kernel(matrix, branch_index, sample_shape):
    ab_flat = _sc_gather(matrix[:, 0], matrix[:, 1],
                         branch_index.astype(jnp.int32))
    ab = ab_flat.reshape(_ROWS, _COLS)
    idx2 = jnp.concatenate([branch_index, branch_index]).reshape(_ROWS, _COLS)
    beta = _tc_sample(ab, idx2.astype(jnp.int32)).reshape(_BATCH)
    return beta / jnp.asarray(sample_shape).astype(jnp.float32)


# speedup vs baseline: 179.3410x; 1.0357x over previous
import functools

import numpy as np
import jax
import jax.numpy as jnp
from jax import lax
from jax.experimental import pallas as pl
from jax.experimental.pallas import tpu as pltpu
from jax.experimental.pallas import tpu_sc as plsc

_BATCH = 16384
_ROWS = 256
_COLS = 128

_ROTS0 = (13, 15, 26, 6)
_ROTS1 = (17, 29, 16, 24)


def _tf_block(k1, k2, x1, x2):
    ks2 = k1 ^ k2 ^ np.uint32(0x1BD11BDA)

    def rotl(x, r):
        return (x << np.uint32(r)) | lax.shift_right_logical(x, np.uint32(32 - r))

    def rounds(a, b, rots):
        for r in rots:
            a = a + b
            b = rotl(b, r)
            b = a ^ b
        return a, b

    a, b = x1 + k1, x2 + k2
    a, b = rounds(a, b, _ROTS0)
    a, b = a + k2, b + ks2 + np.uint32(1)
    a, b = rounds(a, b, _ROTS1)
    a, b = a + ks2, b + k1 + np.uint32(2)
    a, b = rounds(a, b, _ROTS0)
    a, b = a + k1, b + k2 + np.uint32(3)
    a, b = rounds(a, b, _ROTS1)
    a, b = a + k2, b + ks2 + np.uint32(4)
    a, b = rounds(a, b, _ROTS0)
    a, b = a + ks2, b + k1 + np.uint32(5)
    return a, b


def _np_tf_block(k1, k2, x1, x2):
    old = np.seterr(over="ignore")
    try:
        k1, k2, x1, x2 = (np.uint32(v) for v in (k1, k2, x1, x2))
        ks2 = np.uint32(k1 ^ k2 ^ np.uint32(0x1BD11BDA))

        def rotl(x, r):
            return np.uint32(np.uint32(x << np.uint32(r))
                             | np.uint32(x >> np.uint32(32 - r)))

        def rounds(a, b, rots):
            for r in rots:
                a = np.uint32(a + b)
                b = rotl(b, r)
                b = np.uint32(a ^ b)
            return a, b

        a, b = np.uint32(x1 + k1), np.uint32(x2 + k2)
        a, b = rounds(a, b, _ROTS0)
        a, b = np.uint32(a + k2), np.uint32(b + ks2 + np.uint32(1))
        a, b = rounds(a, b, _ROTS1)
        a, b = np.uint32(a + ks2), np.uint32(b + k1 + np.uint32(2))
        a, b = rounds(a, b, _ROTS0)
        a, b = np.uint32(a + k1), np.uint32(b + k2 + np.uint32(3))
        a, b = rounds(a, b, _ROTS1)
        a, b = np.uint32(a + k2), np.uint32(b + ks2 + np.uint32(4))
        a, b = rounds(a, b, _ROTS0)
        a, b = np.uint32(a + ks2), np.uint32(b + k1 + np.uint32(5))
        return a, b
    finally:
        np.seterr(**old)


_KA = _np_tf_block(0, 42, 0, 0)
_KB = _np_tf_block(0, 42, 0, 1)


def _subkey(k1, k2, j):
    z = jnp.zeros_like(k1)
    return _tf_block(k1, k2, z, z + np.uint32(j))


def _bits(k1, k2):
    z = jnp.zeros_like(k1)
    b1, b2 = _tf_block(k1, k2, z, z)
    return b1 ^ b2


def _uniform01(bits):
    fb = lax.shift_right_logical(bits, np.uint32(9)) | np.uint32(0x3F800000)
    return lax.bitcast_convert_type(fb, jnp.float32) - np.float32(1.0)


_NORMAL_LO = np.nextafter(np.float32(-1.0), np.float32(0.0), dtype=np.float32)
_NORMAL_SCALE = np.float32(np.float32(1.0) - _NORMAL_LO)


def _erfinv(x):
    w = -jnp.log1p(-x * x)
    lt = w < np.float32(5.0)
    w1 = w - np.float32(2.5)
    p1 = jnp.full_like(x, np.float32(2.81022636e-08))
    for cc in (3.43273939e-07, -3.5233877e-06, -4.39150654e-06, 0.00021858087,
               -0.00125372503, -0.00417768164, 0.246640727, 1.50140941):
        p1 = np.float32(cc) + p1 * w1
    w2 = jnp.sqrt(w) - np.float32(3.0)
    p2 = jnp.full_like(x, np.float32(-0.000200214257))
    for cc in (0.000100950558, 0.00134934322, -0.00367342844, 0.00573950773,
               -0.0076224613, 0.00943887047, 1.00167406, 2.83297682):
        p2 = np.float32(cc) + p2 * w2
    return jnp.where(lt, p1, p2) * x


def _normal(k1, k2):
    u = _uniform01(_bits(k1, k2)) * _NORMAL_SCALE + _NORMAL_LO
    u = jnp.maximum(_NORMAL_LO, u)
    return np.float32(np.sqrt(2.0)) * _erfinv(u)


_N_LANES = np.int32(_ROWS * _COLS)


def _sample_body(ab_ref, idx_ref, out_ref,
                 lk1_ref, lk2_ref, V_ref, act_ref,
                 ik1_ref, ik2_ref, x_ref, v_ref, iact_ref):
    one = np.float32(1.0)
    m = jnp.maximum(ab_ref[...], np.float32(0.0))
    alpha = jnp.clip(m, np.float32(0.1), np.float32(5.0))
    idx = idx_ref[...].astype(jnp.uint32)

    row = lax.broadcasted_iota(jnp.int32, (_ROWS, _COLS), 0)
    in_a = row < (_ROWS // 2)
    k1 = jnp.where(in_a, np.uint32(_KA[0]), np.uint32(_KB[0])).astype(jnp.uint32)
    k2 = jnp.where(in_a, np.uint32(_KA[1]), np.uint32(_KB[1])).astype(jnp.uint32)
    e1, e2 = _tf_block(k1, k2, jnp.zeros_like(idx), idx)

    boost_mask = alpha >= one
    alpha_b = jnp.where(boost_mask, alpha, alpha + one)
    d = alpha_b - np.float32(1.0 / 3.0)
    c = np.float32(1.0 / 3.0) / lax.sqrt(d)

    lk1, lk2 = _subkey(e1, e2, 0)
    sk1, sk2 = _subkey(e1, e2, 1)
    lk1_ref[...] = lk1
    lk2_ref[...] = lk2
    V_ref[...] = jnp.ones_like(alpha)
    act_ref[...] = jnp.ones((_ROWS, _COLS), jnp.int32)

    def outer_body(_):
        lk1 = lk1_ref[...]
        lk2 = lk2_ref[...]
        active = act_ref[...] != 0
        nk1, nk2 = _subkey(lk1, lk2, 0)
        xk1, xk2 = _subkey(lk1, lk2, 1)
        uk1, uk2 = _subkey(lk1, lk2, 2)

        xn = _normal(*_subkey(xk1, xk2, 1))
        vn = one + xn * c
        ik1_ref[...] = xk1
        ik2_ref[...] = xk2
        x_ref[...] = xn
        v_ref[...] = vn
        iact = active & (vn <= np.float32(0.0))
        iact_ref[...] = iact.astype(jnp.int32)

        def cont_body(_):
            ik1 = ik1_ref[...]
            ik2 = ik2_ref[...]
            iact = iact_ref[...] != 0
            n1, n2 = _subkey(ik1, ik2, 0)
            xn = _normal(*_subkey(n1, n2, 1))
            vn = one + xn * c
            ik1_ref[...] = jnp.where(iact, n1, ik1)
            ik2_ref[...] = jnp.where(iact, n2, ik2)
            x_ref[...] = jnp.where(iact, xn, x_ref[...])
            v_ref[...] = jnp.where(iact, vn, v_ref[...])
            iact = iact & (vn <= np.float32(0.0))
            iact_ref[...] = iact.astype(jnp.int32)
            return jnp.sum(iact.astype(jnp.int32))

        lax.while_loop(lambda nn: nn > 0, cont_body,
                       jnp.sum(iact.astype(jnp.int32)))

        x = x_ref[...]
        v = v_ref[...]
        Xn = x * x
        Vn = (v * v) * v
        Un = _uniform01(_bits(uk1, uk2))

        lk1_ref[...] = jnp.where(active, nk1, lk1)
        lk2_ref[...] = jnp.where(active, nk2, lk2)
        V_ref[...] = jnp.where(active, Vn, V_ref[...])
        cond = (Un >= one - np.float32(0.0331) * (Xn * Xn)) & (
            lax.log(Un) >= Xn * np.float32(0.5)
            + d * ((one - Vn) + lax.log(Vn)))
        active = active & cond
        act_ref[...] = active.astype(jnp.int32)
        return jnp.sum(active.astype(jnp.int32))

    lax.while_loop(lambda n: n > 0, outer_body, _N_LANES)

    u = _uniform01(_bits(sk1, sk2))
    log_samples = lax.log1p(-u)
    log_boost = jnp.where(boost_mask | (log_samples == np.float32(0.0)),
                          np.float32(0.0),
                          log_samples * (one / alpha))
    lg = (lax.log(d) + lax.log(V_ref[...])) + log_boost

    lga = lg[: _ROWS // 2]
    lgb = lg[_ROWS // 2:]
    log_max = jnp.maximum(lga, lgb)
    ga = jnp.exp(lga - log_max)
    gb = jnp.exp(lgb - log_max)
    out_ref[...] = ga / (ga + gb)


def _tc_sample(ab, idx):
    u32s = functools.partial(pltpu.VMEM, (_ROWS, _COLS))
    return pl.pallas_call(
        _sample_body,
        out_shape=jax.ShapeDtypeStruct((_ROWS // 2, _COLS), jnp.float32),
        scratch_shapes=[
            u32s(jnp.uint32), u32s(jnp.uint32),
            u32s(jnp.float32), u32s(jnp.int32),
            u32s(jnp.uint32), u32s(jnp.uint32),
            u32s(jnp.float32), u32s(jnp.float32),
            u32s(jnp.int32),
        ],
    )(ab, idx)


_NC, _NS = 2, 16
_NW = _NC * _NS
_BPW = _BATCH // _NW
_PPW = 2 * _BPW
_CH = 128
_LANES = 16


def _sc_gather(acol, bcol, idx):
    mesh = plsc.VectorSubcoreMesh(core_axis_name="c", subcore_axis_name="s")

    @functools.partial(
        pl.kernel, mesh=mesh,
        compiler_params=pltpu.CompilerParams(use_tc_tiling_on_sc=False),
        out_type=jax.ShapeDtypeStruct((2 * _BATCH,), jnp.float32),
        scratch_types=[
            pltpu.VMEM((_BPW,), jnp.int32),
            pltpu.VMEM((_BPW,), jnp.float32),
            pltpu.VMEM((_BPW,), jnp.float32),
            pltpu.SemaphoreType.DMA,
        ],
    )
    def gather_k(a_hbm, b_hbm, idx_hbm, out_hbm, idx_v, a_v, b_v, sem):
        wid = lax.axis_index("s") * _NC + lax.axis_index("c")
        base = wid * _BPW
        pltpu.sync_copy(idx_hbm.at[pl.ds(base, _BPW)], idx_v)
        copies = []
        for ci in range(_BPW // _CH):
            isl = idx_v.at[pl.ds(ci * _CH, _CH)]
            copies.append(pltpu.async_copy(
                a_hbm.at[isl], a_v.at[pl.ds(ci * _CH, _CH)], sem))
            copies.append(pltpu.async_copy(
                b_hbm.at[isl], b_v.at[pl.ds(ci * _CH, _CH)], sem))
        for cp in copies:
            cp.wait()
        pltpu.sync_copy(a_v, out_hbm.at[pl.ds(base, _BPW)])
        pltpu.sync_copy(b_v, out_hbm.at[pl.ds(_BATCH + base, _BPW)])

    return gather_k(acol, bcol, idx)


def kernel(matrix, branch_index, sample_shape):
    ab_flat = _sc_gather(matrix[:, 0], matrix[:, 1],
                         branch_index.astype(jnp.int32))
    ab = ab_flat.reshape(_ROWS, _COLS)
    idx2 = jnp.concatenate([branch_index, branch_index]).reshape(_ROWS, _COLS)
    beta = _tc_sample(ab, idx2.astype(jnp.int32)).reshape(_BATCH)
    return beta / jnp.asarray(sample_shape).astype(jnp.float32)


# speedup vs baseline: 324.9104x; 1.8117x over previous
import functools

import numpy as np
import jax
import jax.numpy as jnp
from jax import lax
from jax.experimental import pallas as pl
from jax.experimental.pallas import tpu as pltpu
from jax.experimental.pallas import tpu_sc as plsc

_BATCH = 16384
_ROWS = 256
_COLS = 128

_ROTS0 = (13, 15, 26, 6)
_ROTS1 = (17, 29, 16, 24)


def _tf_block(k1, k2, x1, x2):
    ks2 = k1 ^ k2 ^ np.uint32(0x1BD11BDA)

    def rotl(x, r):
        return (x << np.uint32(r)) | lax.shift_right_logical(x, np.uint32(32 - r))

    def rounds(a, b, rots):
        for r in rots:
            a = a + b
            b = rotl(b, r)
            b = a ^ b
        return a, b

    a, b = x1 + k1, x2 + k2
    a, b = rounds(a, b, _ROTS0)
    a, b = a + k2, b + ks2 + np.uint32(1)
    a, b = rounds(a, b, _ROTS1)
    a, b = a + ks2, b + k1 + np.uint32(2)
    a, b = rounds(a, b, _ROTS0)
    a, b = a + k1, b + k2 + np.uint32(3)
    a, b = rounds(a, b, _ROTS1)
    a, b = a + k2, b + ks2 + np.uint32(4)
    a, b = rounds(a, b, _ROTS0)
    a, b = a + ks2, b + k1 + np.uint32(5)
    return a, b


def _np_tf_block(k1, k2, x1, x2):
    old = np.seterr(over="ignore")
    try:
        k1, k2, x1, x2 = (np.uint32(v) for v in (k1, k2, x1, x2))
        ks2 = np.uint32(k1 ^ k2 ^ np.uint32(0x1BD11BDA))

        def rotl(x, r):
            return np.uint32(np.uint32(x << np.uint32(r))
                             | np.uint32(x >> np.uint32(32 - r)))

        def rounds(a, b, rots):
            for r in rots:
                a = np.uint32(a + b)
                b = rotl(b, r)
                b = np.uint32(a ^ b)
            return a, b

        a, b = np.uint32(x1 + k1), np.uint32(x2 + k2)
        a, b = rounds(a, b, _ROTS0)
        a, b = np.uint32(a + k2), np.uint32(b + ks2 + np.uint32(1))
        a, b = rounds(a, b, _ROTS1)
        a, b = np.uint32(a + ks2), np.uint32(b + k1 + np.uint32(2))
        a, b = rounds(a, b, _ROTS0)
        a, b = np.uint32(a + k1), np.uint32(b + k2 + np.uint32(3))
        a, b = rounds(a, b, _ROTS1)
        a, b = np.uint32(a + k2), np.uint32(b + ks2 + np.uint32(4))
        a, b = rounds(a, b, _ROTS0)
        a, b = np.uint32(a + ks2), np.uint32(b + k1 + np.uint32(5))
        return a, b
    finally:
        np.seterr(**old)


_KA = _np_tf_block(0, 42, 0, 0)
_KB = _np_tf_block(0, 42, 0, 1)


def _subkey(k1, k2, j):
    z = jnp.zeros_like(k1)
    return _tf_block(k1, k2, z, z + np.uint32(j))


def _bits(k1, k2):
    z = jnp.zeros_like(k1)
    b1, b2 = _tf_block(k1, k2, z, z)
    return b1 ^ b2


def _uniform01(bits):
    fb = lax.shift_right_logical(bits, np.uint32(9)) | np.uint32(0x3F800000)
    return lax.bitcast_convert_type(fb, jnp.float32) - np.float32(1.0)


_NORMAL_LO = np.nextafter(np.float32(-1.0), np.float32(0.0), dtype=np.float32)
_NORMAL_SCALE = np.float32(np.float32(1.0) - _NORMAL_LO)


def _erfinv(x):
    w = -jnp.log1p(-x * x)
    lt = w < np.float32(5.0)
    w1 = w - np.float32(2.5)
    p1 = jnp.full_like(x, np.float32(2.81022636e-08))
    for cc in (3.43273939e-07, -3.5233877e-06, -4.39150654e-06, 0.00021858087,
               -0.00125372503, -0.00417768164, 0.246640727, 1.50140941):
        p1 = np.float32(cc) + p1 * w1
    w2 = jnp.sqrt(w) - np.float32(3.0)
    p2 = jnp.full_like(x, np.float32(-0.000200214257))
    for cc in (0.000100950558, 0.00134934322, -0.00367342844, 0.00573950773,
               -0.0076224613, 0.00943887047, 1.00167406, 2.83297682):
        p2 = np.float32(cc) + p2 * w2
    return jnp.where(lt, p1, p2) * x


def _normal(k1, k2):
    u = _uniform01(_bits(k1, k2)) * _NORMAL_SCALE + _NORMAL_LO
    u = jnp.maximum(_NORMAL_LO, u)
    return np.float32(np.sqrt(2.0)) * _erfinv(u)


_N_LANES = np.int32(_ROWS * _COLS)


def _sample_body(ab_ref, idx_ref, out_ref,
                 lk1_ref, lk2_ref, V_ref, act_ref,
                 ik1_ref, ik2_ref, x_ref, v_ref, iact_ref):
    one = np.float32(1.0)
    m = jnp.maximum(ab_ref[...], np.float32(0.0))
    alpha = jnp.clip(m, np.float32(0.1), np.float32(5.0))
    idx = idx_ref[...].astype(jnp.uint32)

    row = lax.broadcasted_iota(jnp.int32, (_ROWS, _COLS), 0)
    in_a = row < (_ROWS // 2)
    k1 = jnp.where(in_a, np.uint32(_KA[0]), np.uint32(_KB[0])).astype(jnp.uint32)
    k2 = jnp.where(in_a, np.uint32(_KA[1]), np.uint32(_KB[1])).astype(jnp.uint32)
    e1, e2 = _tf_block(k1, k2, jnp.zeros_like(idx), idx)

    boost_mask = alpha >= one
    alpha_b = jnp.where(boost_mask, alpha, alpha + one)
    d = alpha_b - np.float32(1.0 / 3.0)
    c = np.float32(1.0 / 3.0) / lax.sqrt(d)

    lk1, lk2 = _subkey(e1, e2, 0)
    sk1, sk2 = _subkey(e1, e2, 1)
    lk1_ref[...] = lk1
    lk2_ref[...] = lk2
    V_ref[...] = jnp.ones_like(alpha)
    act_ref[...] = jnp.ones((_ROWS, _COLS), jnp.int32)

    def outer_body(_):
        lk1 = lk1_ref[...]
        lk2 = lk2_ref[...]
        active = act_ref[...] != 0
        nk1, nk2 = _subkey(lk1, lk2, 0)
        xk1, xk2 = _subkey(lk1, lk2, 1)
        uk1, uk2 = _subkey(lk1, lk2, 2)

        xn = _normal(*_subkey(xk1, xk2, 1))
        vn = one + xn * c
        ik1_ref[...] = xk1
        ik2_ref[...] = xk2
        x_ref[...] = xn
        v_ref[...] = vn
        iact = active & (vn <= np.float32(0.0))
        iact_ref[...] = iact.astype(jnp.int32)

        def cont_body(_):
            ik1 = ik1_ref[...]
            ik2 = ik2_ref[...]
            iact = iact_ref[...] != 0
            n1, n2 = _subkey(ik1, ik2, 0)
            xn = _normal(*_subkey(n1, n2, 1))
            vn = one + xn * c
            ik1_ref[...] = jnp.where(iact, n1, ik1)
            ik2_ref[...] = jnp.where(iact, n2, ik2)
            x_ref[...] = jnp.where(iact, xn, x_ref[...])
            v_ref[...] = jnp.where(iact, vn, v_ref[...])
            iact = iact & (vn <= np.float32(0.0))
            iact_ref[...] = iact.astype(jnp.int32)
            return jnp.sum(iact.astype(jnp.int32))

        lax.while_loop(lambda nn: nn > 0, cont_body,
                       jnp.sum(iact.astype(jnp.int32)))

        x = x_ref[...]
        v = v_ref[...]
        Xn = x * x
        Vn = (v * v) * v
        Un = _uniform01(_bits(uk1, uk2))

        lk1_ref[...] = jnp.where(active, nk1, lk1)
        lk2_ref[...] = jnp.where(active, nk2, lk2)
        V_ref[...] = jnp.where(active, Vn, V_ref[...])
        cond = (Un >= one - np.float32(0.0331) * (Xn * Xn)) & (
            lax.log(Un) >= Xn * np.float32(0.5)
            + d * ((one - Vn) + lax.log(Vn)))
        active = active & cond
        act_ref[...] = active.astype(jnp.int32)
        return jnp.sum(active.astype(jnp.int32))

    lax.while_loop(lambda n: n > 0, outer_body, _N_LANES)

    u = _uniform01(_bits(sk1, sk2))
    log_samples = lax.log1p(-u)
    log_boost = jnp.where(boost_mask | (log_samples == np.float32(0.0)),
                          np.float32(0.0),
                          log_samples * (one / alpha))
    lg = (lax.log(d) + lax.log(V_ref[...])) + log_boost

    lga = lg[: _ROWS // 2]
    lgb = lg[_ROWS // 2:]
    log_max = jnp.maximum(lga, lgb)
    ga = jnp.exp(lga - log_max)
    gb = jnp.exp(lgb - log_max)
    out_ref[...] = ga / (ga + gb)


def _tc_sample(ab, idx):
    u32s = functools.partial(pltpu.VMEM, (_ROWS, _COLS))
    return pl.pallas_call(
        _sample_body,
        out_shape=jax.ShapeDtypeStruct((_ROWS // 2, _COLS), jnp.float32),
        scratch_shapes=[
            u32s(jnp.uint32), u32s(jnp.uint32),
            u32s(jnp.float32), u32s(jnp.int32),
            u32s(jnp.uint32), u32s(jnp.uint32),
            u32s(jnp.float32), u32s(jnp.float32),
            u32s(jnp.int32),
        ],
    )(ab, idx)


_NC, _NS = 2, 16
_NW = _NC * _NS
_BPW = _BATCH // _NW
_PPW = 2 * _BPW
_CH = 128
_LANES = 16


def _sc_gather(acol, bcol, idx):
    mesh = plsc.VectorSubcoreMesh(core_axis_name="c", subcore_axis_name="s")

    @functools.partial(
        pl.kernel, mesh=mesh,
        compiler_params=pltpu.CompilerParams(use_tc_tiling_on_sc=False),
        out_type=jax.ShapeDtypeStruct((2 * _BATCH,), jnp.float32),
        scratch_types=[
            pltpu.VMEM((_BPW,), jnp.int32),
            pltpu.VMEM((_BPW,), jnp.float32),
            pltpu.VMEM((_BPW,), jnp.float32),
            pltpu.SemaphoreType.DMA,
        ],
    )
    def gather_k(a_hbm, b_hbm, idx_hbm, out_hbm, idx_v, a_v, b_v, sem):
        wid = lax.axis_index("s") * _NC + lax.axis_index("c")
        base = wid * _BPW
        pltpu.sync_copy(idx_hbm.at[pl.ds(base, _BPW)], idx_v)
        copies = []
        for ci in range(_BPW // _CH):
            isl = idx_v.at[pl.ds(ci * _CH, _CH)]
            copies.append(pltpu.async_copy(
                a_hbm.at[isl], a_v.at[pl.ds(ci * _CH, _CH)], sem))
            copies.append(pltpu.async_copy(
                b_hbm.at[isl], b_v.at[pl.ds(ci * _CH, _CH)], sem))
        for cp in copies:
            cp.wait()
        pltpu.sync_copy(a_v, out_hbm.at[pl.ds(base, _BPW)])
        pltpu.sync_copy(b_v, out_hbm.at[pl.ds(_BATCH + base, _BPW)])

    return gather_k(acol, bcol, idx)


def _extract_body(mT_ref, a_ref, b_ref):
    a_ref[...] = mT_ref[0, :]
    b_ref[...] = mT_ref[1, :]


def _tc_extract(mT):
    n = mT.shape[1]
    return pl.pallas_call(
        _extract_body,
        out_shape=(jax.ShapeDtypeStruct((n,), jnp.float32),
                   jax.ShapeDtypeStruct((n,), jnp.float32)),
    )(mT)


def kernel(matrix, branch_index, sample_shape):
    acol, bcol = _tc_extract(matrix.T)
    ab_flat = _sc_gather(acol, bcol, branch_index.astype(jnp.int32))
    ab = ab_flat.reshape(_ROWS, _COLS)
    idx2 = jnp.concatenate([branch_index, branch_index]).reshape(_ROWS, _COLS)
    beta = _tc_sample(ab, idx2.astype(jnp.int32)).reshape(_BATCH)
    return beta / jnp.asarray(sample_shape).astype(jnp.float32)


# speedup vs baseline: 330.1447x; 1.0161x over previous
import functools

import numpy as np
import jax
import jax.numpy as jnp
from jax import lax
from jax.experimental import pallas as pl
from jax.experimental.pallas import tpu as pltpu
from jax.experimental.pallas import tpu_sc as plsc

_BATCH = 16384
_ROWS = 256
_COLS = 128

_ROTS0 = (13, 15, 26, 6)
_ROTS1 = (17, 29, 16, 24)


def _tf_block(k1, k2, x1, x2):
    ks2 = k1 ^ k2 ^ np.uint32(0x1BD11BDA)

    def rotl(x, r):
        return (x << np.uint32(r)) | lax.shift_right_logical(x, np.uint32(32 - r))

    def rounds(a, b, rots):
        for r in rots:
            a = a + b
            b = rotl(b, r)
            b = a ^ b
        return a, b

    a, b = x1 + k1, x2 + k2
    a, b = rounds(a, b, _ROTS0)
    a, b = a + k2, b + ks2 + np.uint32(1)
    a, b = rounds(a, b, _ROTS1)
    a, b = a + ks2, b + k1 + np.uint32(2)
    a, b = rounds(a, b, _ROTS0)
    a, b = a + k1, b + k2 + np.uint32(3)
    a, b = rounds(a, b, _ROTS1)
    a, b = a + k2, b + ks2 + np.uint32(4)
    a, b = rounds(a, b, _ROTS0)
    a, b = a + ks2, b + k1 + np.uint32(5)
    return a, b


def _np_tf_block(k1, k2, x1, x2):
    old = np.seterr(over="ignore")
    try:
        k1, k2, x1, x2 = (np.uint32(v) for v in (k1, k2, x1, x2))
        ks2 = np.uint32(k1 ^ k2 ^ np.uint32(0x1BD11BDA))

        def rotl(x, r):
            return np.uint32(np.uint32(x << np.uint32(r))
                             | np.uint32(x >> np.uint32(32 - r)))

        def rounds(a, b, rots):
            for r in rots:
                a = np.uint32(a + b)
                b = rotl(b, r)
                b = np.uint32(a ^ b)
            return a, b

        a, b = np.uint32(x1 + k1), np.uint32(x2 + k2)
        a, b = rounds(a, b, _ROTS0)
        a, b = np.uint32(a + k2), np.uint32(b + ks2 + np.uint32(1))
        a, b = rounds(a, b, _ROTS1)
        a, b = np.uint32(a + ks2), np.uint32(b + k1 + np.uint32(2))
        a, b = rounds(a, b, _ROTS0)
        a, b = np.uint32(a + k1), np.uint32(b + k2 + np.uint32(3))
        a, b = rounds(a, b, _ROTS1)
        a, b = np.uint32(a + k2), np.uint32(b + ks2 + np.uint32(4))
        a, b = rounds(a, b, _ROTS0)
        a, b = np.uint32(a + ks2), np.uint32(b + k1 + np.uint32(5))
        return a, b
    finally:
        np.seterr(**old)


_KA = _np_tf_block(0, 42, 0, 0)
_KB = _np_tf_block(0, 42, 0, 1)


def _subkey(k1, k2, j):
    z = jnp.zeros_like(k1)
    return _tf_block(k1, k2, z, z + np.uint32(j))


def _bits(k1, k2):
    z = jnp.zeros_like(k1)
    b1, b2 = _tf_block(k1, k2, z, z)
    return b1 ^ b2


def _uniform01(bits):
    fb = lax.shift_right_logical(bits, np.uint32(9)) | np.uint32(0x3F800000)
    return lax.bitcast_convert_type(fb, jnp.float32) - np.float32(1.0)


_NORMAL_LO = np.nextafter(np.float32(-1.0), np.float32(0.0), dtype=np.float32)
_NORMAL_SCALE = np.float32(np.float32(1.0) - _NORMAL_LO)


def _erfinv(x):
    w = -jnp.log1p(-x * x)
    lt = w < np.float32(5.0)
    w1 = w - np.float32(2.5)
    p1 = jnp.full_like(x, np.float32(2.81022636e-08))
    for cc in (3.43273939e-07, -3.5233877e-06, -4.39150654e-06, 0.00021858087,
               -0.00125372503, -0.00417768164, 0.246640727, 1.50140941):
        p1 = np.float32(cc) + p1 * w1
    w2 = jnp.sqrt(w) - np.float32(3.0)
    p2 = jnp.full_like(x, np.float32(-0.000200214257))
    for cc in (0.000100950558, 0.00134934322, -0.00367342844, 0.00573950773,
               -0.0076224613, 0.00943887047, 1.00167406, 2.83297682):
        p2 = np.float32(cc) + p2 * w2
    return jnp.where(lt, p1, p2) * x


def _normal(k1, k2):
    u = _uniform01(_bits(k1, k2)) * _NORMAL_SCALE + _NORMAL_LO
    u = jnp.maximum(_NORMAL_LO, u)
    return np.float32(np.sqrt(2.0)) * _erfinv(u)


_N_LANES = np.int32(_ROWS * _COLS)


def _prologue(idx):
    row = lax.broadcasted_iota(jnp.int32, (_ROWS, _COLS), 0)
    in_a = row < (_ROWS // 2)
    k1 = jnp.where(in_a, np.uint32(_KA[0]), np.uint32(_KB[0])).astype(jnp.uint32)
    k2 = jnp.where(in_a, np.uint32(_KA[1]), np.uint32(_KB[1])).astype(jnp.uint32)
    e1, e2 = _tf_block(k1, k2, jnp.zeros_like(idx), idx)

    lk1, lk2 = _subkey(e1, e2, 0)
    sk1, sk2 = _subkey(e1, e2, 1)
    nk1, nk2 = _subkey(lk1, lk2, 0)
    xk1, xk2 = _subkey(lk1, lk2, 1)
    uk1, uk2 = _subkey(lk1, lk2, 2)
    xn1 = _normal(*_subkey(xk1, xk2, 1))
    un1 = _uniform01(_bits(uk1, uk2))
    return xn1, un1, xk1, xk2, nk1, nk2, sk1, sk2


def _sample_body(ab_ref, xn1_ref, un1_ref, xk1_ref, xk2_ref,
                 nk1_ref, nk2_ref, sk1_ref, sk2_ref, out_ref,
                 lk1_ref, lk2_ref, V_ref, act_ref,
                 ik1_ref, ik2_ref, x_ref, v_ref, iact_ref):
    one = np.float32(1.0)
    m = jnp.maximum(ab_ref[...], np.float32(0.0))
    alpha = jnp.clip(m, np.float32(0.1), np.float32(5.0))
    sk1 = sk1_ref[...]
    sk2 = sk2_ref[...]

    boost_mask = alpha >= one
    alpha_b = jnp.where(boost_mask, alpha, alpha + one)
    d = alpha_b - np.float32(1.0 / 3.0)
    c = np.float32(1.0 / 3.0) / lax.sqrt(d)

    def one_iteration(active, xn, Un, xk1, xk2, nk1, nk2, lk1, lk2):
        vn = one + xn * c
        ik1_ref[...] = xk1
        ik2_ref[...] = xk2
        x_ref[...] = xn
        v_ref[...] = vn
        iact = active & (vn <= np.float32(0.0))
        iact_ref[...] = iact.astype(jnp.int32)

        def cont_body(_):
            ik1 = ik1_ref[...]
            ik2 = ik2_ref[...]
            iact = iact_ref[...] != 0
            xn = _normal(*_subkey(*_subkey(ik1, ik2, 0), 1))
            n1, n2 = _subkey(ik1, ik2, 0)
            vn = one + xn * c
            ik1_ref[...] = jnp.where(iact, n1, ik1)
            ik2_ref[...] = jnp.where(iact, n2, ik2)
            x_ref[...] = jnp.where(iact, xn, x_ref[...])
            v_ref[...] = jnp.where(iact, vn, v_ref[...])
            iact = iact & (vn <= np.float32(0.0))
            iact_ref[...] = iact.astype(jnp.int32)
            return jnp.sum(iact.astype(jnp.int32))

        lax.while_loop(lambda nn: nn > 0, cont_body,
                       jnp.sum(iact.astype(jnp.int32)))

        x = x_ref[...]
        v = v_ref[...]
        Xn = x * x
        Vn = (v * v) * v

        lk1_ref[...] = jnp.where(active, nk1, lk1)
        lk2_ref[...] = jnp.where(active, nk2, lk2)
        V_ref[...] = jnp.where(active, Vn, V_ref[...])
        cond = (Un >= one - np.float32(0.0331) * (Xn * Xn)) & (
            lax.log(Un) >= Xn * np.float32(0.5)
            + d * ((one - Vn) + lax.log(Vn)))
        active = active & cond
        act_ref[...] = active.astype(jnp.int32)
        return jnp.sum(active.astype(jnp.int32))

    V_ref[...] = jnp.ones_like(alpha)
    all_active = jnp.ones((_ROWS, _COLS), dtype=jnp.bool_)
    n_active = one_iteration(
        all_active, xn1_ref[...], un1_ref[...],
        xk1_ref[...], xk2_ref[...], nk1_ref[...], nk2_ref[...],
        nk1_ref[...], nk2_ref[...])

    def outer_body(_):
        lk1 = lk1_ref[...]
        lk2 = lk2_ref[...]
        active = act_ref[...] != 0
        nk1, nk2 = _subkey(lk1, lk2, 0)
        xk1, xk2 = _subkey(lk1, lk2, 1)
        uk1, uk2 = _subkey(lk1, lk2, 2)
        xn = _normal(*_subkey(xk1, xk2, 1))
        Un = _uniform01(_bits(uk1, uk2))
        return one_iteration(active, xn, Un, xk1, xk2, nk1, nk2, lk1, lk2)

    lax.while_loop(lambda n: n > 0, outer_body, n_active)

    u = _uniform01(_bits(sk1, sk2))
    log_samples = lax.log1p(-u)
    log_boost = jnp.where(boost_mask | (log_samples == np.float32(0.0)),
                          np.float32(0.0),
                          log_samples * (one / alpha))
    lg = (lax.log(d) + lax.log(V_ref[...])) + log_boost

    lga = lg[: _ROWS // 2]
    lgb = lg[_ROWS // 2:]
    log_max = jnp.maximum(lga, lgb)
    ga = jnp.exp(lga - log_max)
    gb = jnp.exp(lgb - log_max)
    out_ref[...] = ga / (ga + gb)


def _tc_sample(ab, pro):
    u32s = functools.partial(pltpu.VMEM, (_ROWS, _COLS))
    return pl.pallas_call(
        _sample_body,
        out_shape=jax.ShapeDtypeStruct((_ROWS // 2, _COLS), jnp.float32),
        scratch_shapes=[
            u32s(jnp.uint32), u32s(jnp.uint32),
            u32s(jnp.float32), u32s(jnp.int32),
            u32s(jnp.uint32), u32s(jnp.uint32),
            u32s(jnp.float32), u32s(jnp.float32),
            u32s(jnp.int32),
        ],
    )(ab, *pro)


_NC, _NS = 2, 16
_NW = _NC * _NS
_BPW = _BATCH // _NW
_PPW = 2 * _BPW
_CH = 128
_LANES = 16


def _sc_gather(acol, bcol, idx):
    mesh = plsc.VectorSubcoreMesh(core_axis_name="c", subcore_axis_name="s")

    @functools.partial(
        pl.kernel, mesh=mesh,
        compiler_params=pltpu.CompilerParams(use_tc_tiling_on_sc=False),
        out_type=jax.ShapeDtypeStruct((2 * _BATCH,), jnp.float32),
        scratch_types=[
            pltpu.VMEM((_BPW,), jnp.int32),
            pltpu.VMEM((_BPW,), jnp.float32),
            pltpu.VMEM((_BPW,), jnp.float32),
            pltpu.SemaphoreType.DMA,
        ],
    )
    def gather_k(a_hbm, b_hbm, idx_hbm, out_hbm, idx_v, a_v, b_v, sem):
        wid = lax.axis_index("s") * _NC + lax.axis_index("c")
        base = wid * _BPW
        pltpu.sync_copy(idx_hbm.at[pl.ds(base, _BPW)], idx_v)
        copies = []
        for ci in range(_BPW // _CH):
            isl = idx_v.at[pl.ds(ci * _CH, _CH)]
            copies.append(pltpu.async_copy(
                a_hbm.at[isl], a_v.at[pl.ds(ci * _CH, _CH)], sem))
            copies.append(pltpu.async_copy(
                b_hbm.at[isl], b_v.at[pl.ds(ci * _CH, _CH)], sem))
        for cp in copies:
            cp.wait()
        pltpu.sync_copy(a_v, out_hbm.at[pl.ds(base, _BPW)])
        pltpu.sync_copy(b_v, out_hbm.at[pl.ds(_BATCH + base, _BPW)])

    return gather_k(acol, bcol, idx)


def _extract_body(mT_ref, idx_ref, a_ref, b_ref, xn1_ref, un1_ref,
                  xk1_ref, xk2_ref, nk1_ref, nk2_ref, sk1_ref, sk2_ref):
    a_ref[...] = mT_ref[0, :]
    b_ref[...] = mT_ref[1, :]
    xn1, un1, xk1, xk2, nk1, nk2, sk1, sk2 = _prologue(
        idx_ref[...].astype(jnp.uint32))
    xn1_ref[...] = xn1
    un1_ref[...] = un1
    xk1_ref[...] = xk1
    xk2_ref[...] = xk2
    nk1_ref[...] = nk1
    nk2_ref[...] = nk2
    sk1_ref[...] = sk1
    sk2_ref[...] = sk2


def _tc_extract(mT, idx2):
    n = mT.shape[1]
    f32s = jax.ShapeDtypeStruct((_ROWS, _COLS), jnp.float32)
    u32s = jax.ShapeDtypeStruct((_ROWS, _COLS), jnp.uint32)
    outs = pl.pallas_call(
        _extract_body,
        out_shape=(jax.ShapeDtypeStruct((n,), jnp.float32),
                   jax.ShapeDtypeStruct((n,), jnp.float32),
                   f32s, f32s, u32s, u32s, u32s, u32s, u32s, u32s),
    )(mT, idx2)
    return outs[0], outs[1], outs[2:]


def kernel(matrix, branch_index, sample_shape):
    idx2 = jnp.concatenate([branch_index, branch_index]).reshape(_ROWS, _COLS)
    acol, bcol, pro = _tc_extract(matrix.T, idx2.astype(jnp.int32))
    ab_flat = _sc_gather(acol, bcol, branch_index.astype(jnp.int32))
    ab = ab_flat.reshape(_ROWS, _COLS)
    beta = _tc_sample(ab, pro).reshape(_BATCH)
    return beta / jnp.asarray(sample_shape).astype(jnp.float32)
